# baseline (speedup 1.0000x reference)
import functools

import jax
import jax.numpy as jnp
from jax import lax
from jax.experimental import pallas as pl
from jax.experimental.pallas import tpu as pltpu
from jax.experimental.pallas import tpu_sc as plsc

_B, _S, _LVL, _EMB = 1024, 50, 8, 64
_LANES = 16
_NW = 32
_BPW = _B // _NW
_CROWS = _BPW * _LVL
_IDXR = _CROWS // 128
_V = 1000000
_NBLK = _V // 128
_VTAIL = _V - _NBLK * 128
_GCOLS = 512
_GBLK = _GCOLS // 128
_NGRP = _NBLK // _GBLK
_GSLOTS = (_NGRP + _NW - 1) // _NW


def _tr_body(tv_hbm, tail_hbm, out_hbm, blk_v, tr_v, trf_v, tail_v,
             gsem0, gsem1, osem00, osem01, osem10, osem11):
    info = plsc.get_sparse_core_info()
    nc = info.num_cores
    wid = lax.axis_index("s") * nc + lax.axis_index("c")
    gsems = (gsem0, gsem1)
    osems = ((osem00, osem01), (osem10, osem11))
    iota = lax.broadcasted_iota(jnp.int32, (_LANES,), 0)

    def fire(g, buf):
        @pl.when(g < _NGRP)
        def _():
            pltpu.async_copy(
                tv_hbm.at[:, pl.ds(g * _GCOLS, _GCOLS)], blk_v.at[buf],
                gsems[buf],
            )

    def drain(g, buf):
        @pl.when(g < _NGRP)
        def _():
            pltpu.make_async_copy(
                tv_hbm.at[:, pl.ds(g * _GCOLS, _GCOLS)], blk_v.at[buf],
                gsems[buf],
            ).wait()

    def handle_group(g, buf):
        @pl.when(g < _NGRP)
        def _():
            for sb in range(_GBLK):
                tf = sb % 2
                if sb >= 2:
                    pltpu.make_async_copy(
                        trf_v.at[buf, tf], out_hbm.at[pl.ds(0, 8192)],
                        osems[buf][tf],
                    ).wait()
                else:
                    @pl.when(g >= 2 * _NW)
                    def _w():
                        pltpu.make_async_copy(
                            trf_v.at[buf, tf], out_hbm.at[pl.ds(0, 8192)],
                            osems[buf][tf],
                        ).wait()

                def cgrp(cg, carry, sb=sb):
                    for cc in range(2):
                        c = cg * 2 + cc
                        col = iota * 0 + c
                        for k8 in range(8):
                            vals = blk_v[
                                buf, c, pl.ds(sb * 128 + k8 * _LANES, _LANES)
                            ]
                            plsc.store_scatter(
                                tr_v, [k8 * _LANES + iota, col], vals
                            )
                    return carry

                lax.fori_loop(0, _EMB // 2, cgrp, 0)

                def rcpy(rg, carry, tf=tf):
                    for rr in range(4):
                        r = rg * 4 + rr
                        for k4 in range(4):
                            trf_v[buf, tf,
                                  pl.ds(r * _EMB + k4 * _LANES, _LANES)] = (
                                tr_v[r, pl.ds(k4 * _LANES, _LANES)]
                            )
                    return carry

                lax.fori_loop(0, 32, rcpy, 0)
                pltpu.async_copy(
                    trf_v.at[buf, tf],
                    out_hbm.at[pl.ds((g * _GBLK + sb) * 8192, 8192)],
                    osems[buf][tf],
                )

    fire(wid, 0)

    def pair(p, carry):
        for half in range(2):
            k = 2 * p + half
            buf = half
            g = k * _NW + wid
            fire(g + _NW, 1 - buf)
            drain(g, buf)
            handle_group(g, buf)
        return carry

    lax.fori_loop(0, (_GSLOTS + 1) // 2, pair, 0)
    for buf in range(2):
        for tf in range(2):
            pltpu.make_async_copy(
                trf_v.at[buf, tf], out_hbm.at[pl.ds(0, 8192)],
                osems[buf][tf],
            ).wait()

    @pl.when(wid == 4)
    def _():
        pltpu.sync_copy(tail_hbm, tail_v)
        pltpu.sync_copy(
            tail_v, out_hbm.at[pl.ds(_NBLK * 128 * _EMB, _VTAIL * _EMB)]
        )


@jax.jit
def _sc_transpose(table):
    tv = table.T
    tail = lax.slice(table, (_NBLK * 128, 0), (_V, _EMB)).reshape(_VTAIL * _EMB)
    run = functools.partial(
        pl.kernel,
        out_type=jax.ShapeDtypeStruct((_V * _EMB,), jnp.float32),
        mesh=plsc.VectorSubcoreMesh(core_axis_name="c", subcore_axis_name="s"),
        scratch_types=[
            pltpu.VMEM((2, _EMB, _GCOLS), jnp.float32),
            pltpu.VMEM((128, _EMB + 1), jnp.float32),
            pltpu.VMEM((2, 2, 8192), jnp.float32),
            pltpu.VMEM((_VTAIL * _EMB,), jnp.float32),
            pltpu.SemaphoreType.DMA,
            pltpu.SemaphoreType.DMA,
            pltpu.SemaphoreType.DMA,
            pltpu.SemaphoreType.DMA,
            pltpu.SemaphoreType.DMA,
            pltpu.SemaphoreType.DMA,
        ],
        compiler_params=pltpu.CompilerParams(
            use_tc_tiling_on_sc=True, needs_layout_passes=False
        ),
    )(_tr_body)
    return run(tv, tail).reshape(_V, _EMB)


def _sc_body(idx_hbm, w_hbm, table_hbm, out_hbm, idx_v, rows_v, out_v, wv,
             gsem0, gsem1, isem0, isem1, osem0, osem1):
    info = plsc.get_sparse_core_info()
    nc = info.num_cores
    wid = lax.axis_index("s") * nc + lax.axis_index("c")
    b0 = wid * _BPW
    gsems = (gsem0, gsem1)
    isems = (isem0, isem1)
    osems = (osem0, osem1)

    pltpu.sync_copy(w_hbm, wv)
    w = wv[...]
    ws = [w[l] for l in range(_LVL)]
    m = ws[0]
    for l in range(1, _LVL):
        m = jnp.maximum(m, ws[l])
    e = jnp.exp(w - m)
    es = [e[l] for l in range(_LVL)]
    s_sum = es[0]
    for l in range(1, _LVL):
        s_sum = s_sum + es[l]
    alpha = e / s_sum
    a = [alpha[l] for l in range(_LVL)]

    iota = lax.broadcasted_iota(jnp.int32, (_LANES,), 0)

    nrounds = _S // 2

    def prefetch_idx(r, buf):
        @pl.when(r < nrounds)
        def _():
            pltpu.async_copy(
                idx_hbm.at[wid, pl.ds(2 * r, 2)], idx_v.at[buf], isems[buf]
            )

    def fire_rows(r, buf):
        @pl.when(r < nrounds)
        def _():
            pltpu.make_async_copy(
                idx_hbm.at[wid, pl.ds(2 * r, 2)], idx_v.at[buf], isems[buf]
            ).wait()
            for h in range(2):
                for j in range(_IDXR):
                    pltpu.async_copy(
                        table_hbm.at[idx_v.at[buf, h, j]],
                        rows_v.at[buf, pl.ds((h * _IDXR + j) * 128, 128)],
                        gsems[buf],
                    )

    def drain(buf):
        for h in range(2):
            for j in range(_IDXR):
                pltpu.make_async_copy(
                    table_hbm.at[idx_v.at[buf, h, j]],
                    rows_v.at[buf, pl.ds((h * _IDXR + j) * 128, 128)],
                    gsems[buf],
                ).wait()

    def wait_out(r, buf):
        for h in range(2):
            pltpu.make_async_copy(
                out_v.at[buf, h, :, pl.ds(0, _BPW)],
                out_hbm.at[2 * r + h, :, pl.ds(b0, _BPW)],
                osems[buf],
            ).wait()

    def combine(r, buf):
        for h in range(2):
            def tok2(i, c, h=h):
                for tt in range(2):
                    bb = i * 2 + tt
                    rbase = (h * _BPW + bb) * _LVL
                    col = iota * 0 + bb
                    for j in range(_EMB // _LANES):
                        sl = pl.ds(j * _LANES, _LANES)
                        acc = a[0] * rows_v[buf, rbase, sl]
                        for l in range(1, _LVL):
                            acc = acc + a[l] * rows_v[buf, rbase + l, sl]
                        plsc.store_scatter(
                            out_v.at[buf, h], [j * _LANES + iota, col], acc
                        )
                return c

            lax.fori_loop(0, _BPW // 2, tok2, 0)
            pltpu.async_copy(
                out_v.at[buf, h, :, pl.ds(0, _BPW)],
                out_hbm.at[2 * r + h, :, pl.ds(b0, _BPW)],
                osems[buf],
            )

    prefetch_idx(0, 0)
    prefetch_idx(1, 1)
    fire_rows(0, 0)

    def round2(p, carry):
        r0 = 2 * p
        fire_rows(r0 + 1, 1)
        drain(0)
        prefetch_idx(r0 + 2, 0)

        @pl.when(r0 >= 2)
        def _():
            wait_out(r0 - 2, 0)

        combine(r0, 0)
        fire_rows(r0 + 2, 0)
        drain(1)
        prefetch_idx(r0 + 3, 1)

        @pl.when(r0 >= 2)
        def _():
            wait_out(r0 - 1, 1)

        combine(r0 + 1, 1)
        return carry

    lax.fori_loop(0, nrounds // 2, round2, 0)
    drain(0)
    wait_out(nrounds - 3, 0)
    combine(nrounds - 1, 0)
    wait_out(nrounds - 2, 1)
    wait_out(nrounds - 1, 0)


@jax.jit
def _sc_gather_combine(idx, w_pad, table):
    run = functools.partial(
        pl.kernel,
        out_type=jax.ShapeDtypeStruct((_S, _EMB, _B), jnp.float32),
        mesh=plsc.VectorSubcoreMesh(core_axis_name="c", subcore_axis_name="s"),
        scratch_types=[
            pltpu.VMEM((2, 2, _IDXR, 128), jnp.int32),
            pltpu.VMEM((2, 2 * _CROWS, _EMB), jnp.float32),
            pltpu.VMEM((2, 2, _EMB, _BPW + 1), jnp.float32),
            pltpu.VMEM((_LANES,), jnp.float32),
            pltpu.SemaphoreType.DMA,
            pltpu.SemaphoreType.DMA,
            pltpu.SemaphoreType.DMA,
            pltpu.SemaphoreType.DMA,
            pltpu.SemaphoreType.DMA,
            pltpu.SemaphoreType.DMA,
        ],
        compiler_params=pltpu.CompilerParams(
            use_tc_tiling_on_sc=False, needs_layout_passes=False
        ),
    )(_sc_body)
    return run(idx, w_pad, table)


def kernel(croutes, tailcs, rc_cid_emb, rc_weight):
    del tailcs
    idx = (
        croutes.reshape(_NW, _BPW, _S, _LVL)
        .transpose(0, 2, 1, 3)
        .reshape(_NW, _S, _IDXR, 128)
    )
    w_pad = jnp.concatenate(
        [rc_weight.astype(jnp.float32),
         jnp.full((_LANES - _LVL,), -jnp.inf, dtype=jnp.float32)]
    )
    out_phys = _sc_gather_combine(idx, w_pad, _sc_transpose(rc_cid_emb))
    return out_phys.transpose(2, 0, 1)


# speedup vs baseline: 2.0836x; 2.0836x over previous
import functools

import jax
import jax.numpy as jnp
from jax import lax
from jax.experimental import pallas as pl
from jax.experimental.pallas import tpu as pltpu
from jax.experimental.pallas import tpu_sc as plsc

_B, _S, _LVL, _EMB = 1024, 50, 8, 64
_LANES = 16
_NW = 32
_BPW = _B // _NW
_CROWS = _BPW * _LVL
_IDXR = _CROWS // 128


def _sc_body(idx_hbm, w_hbm, table_hbm, out_hbm, idx_v, rows_v, out_v, wv,
             gsem0, gsem1, isem0, isem1, osem0, osem1):
    info = plsc.get_sparse_core_info()
    nc = info.num_cores
    wid = lax.axis_index("s") * nc + lax.axis_index("c")
    b0 = wid * _BPW
    gsems = (gsem0, gsem1)
    isems = (isem0, isem1)
    osems = (osem0, osem1)

    pltpu.sync_copy(w_hbm, wv)
    w = wv[...]
    ws = [w[l] for l in range(_LVL)]
    m = ws[0]
    for l in range(1, _LVL):
        m = jnp.maximum(m, ws[l])
    e = jnp.exp(w - m)
    es = [e[l] for l in range(_LVL)]
    s_sum = es[0]
    for l in range(1, _LVL):
        s_sum = s_sum + es[l]
    alpha = e / s_sum
    a = [alpha[l] for l in range(_LVL)]

    iota = lax.broadcasted_iota(jnp.int32, (_LANES,), 0)

    nrounds = _S // 2

    def prefetch_idx(r, buf):
        @pl.when(r < nrounds)
        def _():
            pltpu.async_copy(
                idx_hbm.at[wid, pl.ds(2 * r, 2)], idx_v.at[buf], isems[buf]
            )

    def fire_rows(r, buf):
        @pl.when(r < nrounds)
        def _():
            pltpu.make_async_copy(
                idx_hbm.at[wid, pl.ds(2 * r, 2)], idx_v.at[buf], isems[buf]
            ).wait()
            for h in range(2):
                for j in range(_IDXR):
                    pltpu.async_copy(
                        table_hbm.at[idx_v.at[buf, h, j]],
                        rows_v.at[buf, pl.ds((h * _IDXR + j) * 128, 128)],
                        gsems[buf],
                    )

    def drain(buf):
        for h in range(2):
            for j in range(_IDXR):
                pltpu.make_async_copy(
                    table_hbm.at[idx_v.at[buf, h, j]],
                    rows_v.at[buf, pl.ds((h * _IDXR + j) * 128, 128)],
                    gsems[buf],
                ).wait()

    def wait_out(r, buf):
        for h in range(2):
            pltpu.make_async_copy(
                out_v.at[buf, h, :, pl.ds(0, _BPW)],
                out_hbm.at[2 * r + h, :, pl.ds(b0, _BPW)],
                osems[buf],
            ).wait()

    def combine(r, buf):
        for h in range(2):
            def tok2(i, c, h=h):
                for tt in range(2):
                    bb = i * 2 + tt
                    rbase = (h * _BPW + bb) * _LVL
                    col = iota * 0 + bb
                    for j in range(_EMB // _LANES):
                        sl = pl.ds(j * _LANES, _LANES)
                        acc = a[0] * rows_v[buf, rbase, sl]
                        for l in range(1, _LVL):
                            acc = acc + a[l] * rows_v[buf, rbase + l, sl]
                        plsc.store_scatter(
                            out_v.at[buf, h], [j * _LANES + iota, col], acc
                        )
                return c

            lax.fori_loop(0, _BPW // 2, tok2, 0)
            pltpu.async_copy(
                out_v.at[buf, h, :, pl.ds(0, _BPW)],
                out_hbm.at[2 * r + h, :, pl.ds(b0, _BPW)],
                osems[buf],
            )

    prefetch_idx(0, 0)
    prefetch_idx(1, 1)
    fire_rows(0, 0)

    def round2(p, carry):
        r0 = 2 * p
        fire_rows(r0 + 1, 1)
        drain(0)
        prefetch_idx(r0 + 2, 0)

        @pl.when(r0 >= 2)
        def _():
            wait_out(r0 - 2, 0)

        combine(r0, 0)
        fire_rows(r0 + 2, 0)
        drain(1)
        prefetch_idx(r0 + 3, 1)

        @pl.when(r0 >= 2)
        def _():
            wait_out(r0 - 1, 1)

        combine(r0 + 1, 1)
        return carry

    lax.fori_loop(0, nrounds // 2, round2, 0)
    drain(0)
    wait_out(nrounds - 3, 0)
    combine(nrounds - 1, 0)
    wait_out(nrounds - 2, 1)
    wait_out(nrounds - 1, 0)


@jax.jit
def _sc_gather_combine(idx, w_pad, table):
    run = functools.partial(
        pl.kernel,
        out_type=jax.ShapeDtypeStruct((_S, _EMB, _B), jnp.float32),
        mesh=plsc.VectorSubcoreMesh(core_axis_name="c", subcore_axis_name="s"),
        scratch_types=[
            pltpu.VMEM((2, 2, _IDXR, 128), jnp.int32),
            pltpu.VMEM((2, 2 * _CROWS, _EMB), jnp.float32),
            pltpu.VMEM((2, 2, _EMB, _BPW + 1), jnp.float32),
            pltpu.VMEM((_LANES,), jnp.float32),
            pltpu.SemaphoreType.DMA,
            pltpu.SemaphoreType.DMA,
            pltpu.SemaphoreType.DMA,
            pltpu.SemaphoreType.DMA,
            pltpu.SemaphoreType.DMA,
            pltpu.SemaphoreType.DMA,
        ],
        compiler_params=pltpu.CompilerParams(
            use_tc_tiling_on_sc=False, needs_layout_passes=False
        ),
    )(_sc_body)
    return run(idx, w_pad, table)


def kernel(croutes, tailcs, rc_cid_emb, rc_weight):
    del tailcs
    idx = (
        croutes.reshape(_NW, _BPW, _S, _LVL)
        .transpose(0, 2, 1, 3)
        .reshape(_NW, _S, _IDXR, 128)
    )
    w_pad = jnp.concatenate(
        [rc_weight.astype(jnp.float32),
         jnp.full((_LANES - _LVL,), -jnp.inf, dtype=jnp.float32)]
    )
    out_phys = _sc_gather_combine(idx, w_pad, rc_cid_emb)
    return out_phys.transpose(2, 0, 1)
